# Initial kernel scaffold; baseline (speedup 1.0000x reference)
import functools

import jax
import jax.numpy as jnp
from jax import lax
from jax.experimental import pallas as pl
from jax.experimental.pallas import tpu as pltpu
from jax.experimental.pallas import tpu_sc as plsc

N = 10000
E = 320000
D = 128
NC = 2
NS = 16
K = 128
CHUNKS = 157
EPT = CHUNKS * K
EPAD = EPT * NS
ROWS_PER_TILE = N // NS
BM = 1000


def _matmul_body(x_ref, w_ref, o_ref):
    o_ref[0] = jnp.dot(x_ref[...], w_ref[0], preferred_element_type=jnp.float32)


_matmul = pl.pallas_call(
    _matmul_body,
    grid=(2, N // BM),
    in_specs=[
        pl.BlockSpec((BM, D), lambda g, i: (i, 0)),
        pl.BlockSpec((1, D, D), lambda g, i: (g, 0, 0)),
    ],
    out_specs=pl.BlockSpec((1, BM, D), lambda g, i: (g, i, 0)),
    out_shape=jax.ShapeDtypeStruct((2, N, D), jnp.float32),
)


_sc_mesh = plsc.VectorSubcoreMesh(core_axis_name="c", subcore_axis_name="s")


@functools.partial(
    pl.kernel,
    out_type=jax.ShapeDtypeStruct((NC, N, D), jnp.float32),
    mesh=_sc_mesh,
    scratch_types=[
        pltpu.VMEM((CHUNKS, K), jnp.int32),
        pltpu.VMEM((CHUNKS, K), jnp.int32),
        pltpu.VMEM((CHUNKS, K), jnp.float32),
        pltpu.VMEM((K, D), jnp.float32),
        pltpu.VMEM_SHARED((N, D), jnp.float32),
        pltpu.SemaphoreType.DMA,
    ],
)
def _spmm_kernel(sup_hbm, rows_hbm, cols_hbm, vals_hbm, out_hbm,
                 rows_i, cols_i, vals_v, gbuf, acc, sem):
    c = lax.axis_index("c")
    s = lax.axis_index("s")

    zero16 = jnp.zeros((16,), jnp.float32)

    def zbody(i, _):
        for j in range(D // 16):
            gbuf[i, pl.ds(j * 16, 16)] = zero16
        return 0

    lax.fori_loop(0, K, zbody, 0)
    base = s * ROWS_PER_TILE
    for t in range(4):
        pltpu.sync_copy(gbuf, acc.at[pl.ds(base + t * K, K), :])
    pltpu.sync_copy(gbuf.at[pl.ds(0, ROWS_PER_TILE - 4 * K), :],
                    acc.at[pl.ds(base + 4 * K, ROWS_PER_TILE - 4 * K), :])
    plsc.subcore_barrier()

    pltpu.sync_copy(rows_hbm.at[c, s], rows_i)
    pltpu.sync_copy(cols_hbm.at[c, s], cols_i)
    pltpu.sync_copy(vals_hbm.at[c, s], vals_v)

    def chunk_body(t, _):
        pltpu.async_copy(sup_hbm.at[c].at[cols_i.at[t]], gbuf, sem).wait()

        def scale_body(i, _):
            v = vals_v[t, i]
            for j in range(D // 16):
                sl = pl.ds(j * 16, 16)
                gbuf[i, sl] = gbuf[i, sl] * v
            return 0

        lax.fori_loop(0, K, scale_body, 0)

        pltpu.sync_copy(gbuf, acc.at[rows_i.at[t]], add=True)
        return 0

    lax.fori_loop(0, CHUNKS, chunk_body, 0)

    plsc.subcore_barrier()
    pltpu.sync_copy(acc.at[pl.ds(base, ROWS_PER_TILE), :],
                    out_hbm.at[c, pl.ds(base, ROWS_PER_TILE), :])


def _combine_body(p_ref, b_ref, o_ref):
    o_ref[...] = p_ref[0] + p_ref[1] + b_ref[...]


_combine = pl.pallas_call(
    _combine_body,
    grid=(N // BM,),
    in_specs=[
        pl.BlockSpec((2, BM, D), lambda i: (0, i, 0)),
        pl.BlockSpec((1, D), lambda i: (0, 0)),
    ],
    out_specs=pl.BlockSpec((BM, D), lambda i: (i, 0)),
    out_shape=jax.ShapeDtypeStruct((N, D), jnp.float32),
)


def _prep_idx(a):
    a = a.astype(jnp.int32)
    return jnp.pad(a, (0, EPAD - E)).reshape(NS, CHUNKS, K)


def _prep_val(a):
    return jnp.pad(a.astype(jnp.float32), (0, EPAD - E)).reshape(NS, CHUNKS, K)


def kernel(input, weight_1, weight_2, bias,
           adj0_rows, adj0_cols, adj0_vals,
           adj1_rows, adj1_cols, adj1_vals):
    w = jnp.stack([weight_1, weight_2])
    sup = _matmul(input, w)
    rows = jnp.stack([_prep_idx(adj0_rows), _prep_idx(adj1_rows)])
    cols = jnp.stack([_prep_idx(adj0_cols), _prep_idx(adj1_cols)])
    vals = jnp.stack([_prep_val(adj0_vals), _prep_val(adj1_vals)])
    partial = _spmm_kernel(sup, rows, cols, vals)
    return _combine(partial, bias.reshape(1, D))


# speedup vs baseline: 4.7988x; 4.7988x over previous
import functools

import jax
import jax.numpy as jnp
from jax import lax
from jax.experimental import pallas as pl
from jax.experimental.pallas import tpu as pltpu
from jax.experimental.pallas import tpu_sc as plsc

N = 10000
E = 320000
D = 128
NC = 2
NS = 16
K = 128
CHUNKS = 157
EPT = CHUNKS * K
EPAD = EPT * NS
RPT = 624
BM = 1000


def _matmul_body(x_ref, w_ref, o_ref):
    o_ref[0] = jnp.dot(x_ref[...], w_ref[0],
                       preferred_element_type=jnp.float32)


_matmul = pl.pallas_call(
    _matmul_body,
    grid=(2, N // BM),
    in_specs=[
        pl.BlockSpec((BM, D), lambda g, i: (i, 0)),
        pl.BlockSpec((1, D, D), lambda g, i: (g, 0, 0)),
    ],
    out_specs=pl.BlockSpec((1, BM, D), lambda g, i: (g, i, 0)),
    out_shape=jax.ShapeDtypeStruct((2, N, D), jnp.float32),
)


_sc_mesh = plsc.VectorSubcoreMesh(core_axis_name="c", subcore_axis_name="s")


@functools.partial(
    pl.kernel,
    out_type=jax.ShapeDtypeStruct((NC, N, D), jnp.float32),
    mesh=_sc_mesh,
    scratch_types=[
        pltpu.VMEM((2, K), jnp.int32),
        pltpu.VMEM((1, K), jnp.float32),
        pltpu.VMEM((K, D), jnp.float32),
        pltpu.VMEM_SHARED((N, D), jnp.float32),
        pltpu.SemaphoreType.DMA,
    ],
)
def _spmm_kernel(sup_hbm, idx_hbm, vals_hbm, out_hbm, edg, val, gbuf, acc, sem):
    c = lax.axis_index("c")
    s = lax.axis_index("s")

    zero16 = jnp.zeros((16,), jnp.float32)

    def zbody(i, _):
        for j in range(D // 16):
            gbuf[i, pl.ds(j * 16, 16)] = zero16
        return 0

    lax.fori_loop(0, K, zbody, 0)
    base = s * RPT
    for t in range(4):
        pltpu.sync_copy(gbuf, acc.at[pl.ds(base + t * K, K), :])
    pltpu.sync_copy(gbuf.at[pl.ds(0, RPT - 4 * K), :],
                    acc.at[pl.ds(base + 4 * K, RPT - 4 * K), :])

    @pl.when(s == NS - 1)
    def _zero_tail():
        pltpu.sync_copy(gbuf.at[pl.ds(0, N - NS * RPT), :],
                        acc.at[pl.ds(NS * RPT, N - NS * RPT), :])

    plsc.subcore_barrier()

    def chunk_body(t, _):
        pltpu.sync_copy(idx_hbm.at[c, s, t], edg)
        pltpu.sync_copy(vals_hbm.at[c, s, t], val)

        pltpu.async_copy(sup_hbm.at[c].at[edg.at[1]], gbuf, sem).wait()

        def scale_body(gr, _):
            vv = val[0, pl.ds(gr * 16, 16)]
            for l in range(16):
                v = vv[l]
                i = gr * 16 + l
                for j in range(D // 16):
                    sl = pl.ds(j * 16, 16)
                    gbuf[i, sl] = gbuf[i, sl] * v
            return 0

        lax.fori_loop(0, K // 16, scale_body, 0)

        pltpu.sync_copy(gbuf, acc.at[edg.at[0]], add=True)
        return 0

    lax.fori_loop(0, CHUNKS, chunk_body, 0)

    plsc.subcore_barrier()
    pltpu.sync_copy(acc.at[pl.ds(base, RPT), :],
                    out_hbm.at[c, pl.ds(base, RPT), :])

    @pl.when(s == NS - 1)
    def _drain_tail():
        pltpu.sync_copy(acc.at[pl.ds(NS * RPT, N - NS * RPT), :],
                        out_hbm.at[c, pl.ds(NS * RPT, N - NS * RPT), :])


def _combine_body(p_ref, b_ref, o_ref):
    o_ref[...] = p_ref[0] + p_ref[1] + b_ref[...]


_combine = pl.pallas_call(
    _combine_body,
    grid=(N // BM,),
    in_specs=[
        pl.BlockSpec((2, BM, D), lambda i: (0, i, 0)),
        pl.BlockSpec((1, D), lambda i: (0, 0)),
    ],
    out_specs=pl.BlockSpec((BM, D), lambda i: (i, 0)),
    out_shape=jax.ShapeDtypeStruct((N, D), jnp.float32),
)


def _pad_rs(a):
    return jnp.pad(a, (0, EPAD - E)).reshape(NS, CHUNKS, K)


def _prep_idx(rows, cols):
    return jnp.stack([_pad_rs(rows.astype(jnp.int32)),
                      _pad_rs(cols.astype(jnp.int32))], axis=2)


def _prep_val(vals):
    return _pad_rs(vals.astype(jnp.float32))[:, :, None, :]


def kernel(input, weight_1, weight_2, bias,
           adj0_rows, adj0_cols, adj0_vals,
           adj1_rows, adj1_cols, adj1_vals):
    w = jnp.stack([weight_1, weight_2])
    sup = _matmul(input, w)
    idx = jnp.stack([_prep_idx(adj0_rows, adj0_cols),
                     _prep_idx(adj1_rows, adj1_cols)])
    vals = jnp.stack([_prep_val(adj0_vals), _prep_val(adj1_vals)])
    partial = _spmm_kernel(sup, idx, vals)
    return _combine(partial, bias.reshape(1, D))
